# baseline (speedup 1.0000x reference)
import functools

import jax
import jax.numpy as jnp
from jax import lax
from jax.experimental import pallas as pl
from jax.experimental.pallas import tpu as pltpu
from jax.experimental.pallas import tpu_sc as plsc

BATCH = 1024
SEQ = 200
EMBED_DIM = 64
B = BATCH * SEQ


def _make_gather():
    info = plsc.get_sparse_core_info()
    nc, ns = info.num_cores, info.num_subcores
    nw = nc * ns
    b_per_w = B // nw
    chunk = 800
    n_chunks = b_per_w // chunk

    mesh = plsc.VectorSubcoreMesh(core_axis_name="c", subcore_axis_name="s")

    @functools.partial(
        pl.kernel,
        mesh=mesh,
        compiler_params=pltpu.CompilerParams(use_tc_tiling_on_sc=False),
        out_type=jax.ShapeDtypeStruct((B, EMBED_DIM), jnp.float32),
        scratch_types=[
            pltpu.VMEM((chunk,), jnp.int32),
            pltpu.VMEM((chunk, EMBED_DIM), jnp.float32),
            pltpu.SemaphoreType.DMA,
        ],
    )
    def gather_kernel(idx_hbm, table_hbm, out_hbm, idx_v, rows_v, sem):
        wid = lax.axis_index("s") * nc + lax.axis_index("c")

        def step(j, carry):
            base = wid * b_per_w + j * chunk
            pltpu.sync_copy(idx_hbm.at[pl.ds(base, chunk)], idx_v)
            pltpu.async_copy(table_hbm.at[idx_v], rows_v, sem).wait()
            pltpu.sync_copy(rows_v, out_hbm.at[pl.ds(base, chunk)])
            return carry

        lax.fori_loop(0, n_chunks, step, 0)

    return gather_kernel


_gather = _make_gather()


@jax.jit
def kernel(inp_ids, table):
    flat = inp_ids.reshape(B)
    out = _gather(flat, table)
    return out.reshape(BATCH, SEQ, EMBED_DIM)


# speedup vs baseline: 1.0041x; 1.0041x over previous
import functools

import jax
import jax.numpy as jnp
from jax import lax
from jax.experimental import pallas as pl
from jax.experimental.pallas import tpu as pltpu
from jax.experimental.pallas import tpu_sc as plsc

BATCH = 1024
SEQ = 200
EMBED_DIM = 64
B = BATCH * SEQ


def _make_gather():
    info = plsc.get_sparse_core_info()
    nc, ns = info.num_cores, info.num_subcores
    nw = nc * ns
    b_per_w = B // nw
    chunk = 800
    n_chunks = b_per_w // chunk
    n_buf = 2

    mesh = plsc.VectorSubcoreMesh(core_axis_name="c", subcore_axis_name="s")

    @functools.partial(
        pl.kernel,
        mesh=mesh,
        compiler_params=pltpu.CompilerParams(use_tc_tiling_on_sc=False),
        out_type=jax.ShapeDtypeStruct((B, EMBED_DIM), jnp.float32),
        scratch_types=[
            pltpu.VMEM((n_chunks, chunk), jnp.int32),
            pltpu.VMEM((n_buf, chunk, EMBED_DIM), jnp.float32),
            pltpu.SemaphoreType.DMA,
            pltpu.SemaphoreType.DMA((n_buf,)),
            pltpu.SemaphoreType.DMA((n_buf,)),
        ],
    )
    def gather_kernel(idx_hbm, table_hbm, out_hbm, idx_v, rows_v, isem, gsem, wsem):
        wid = lax.axis_index("s") * nc + lax.axis_index("c")
        base = wid * b_per_w

        idx_copies = [
            pltpu.async_copy(
                idx_hbm.at[pl.ds(base + j * chunk, chunk)], idx_v.at[j], isem
            )
            for j in range(n_chunks)
        ]
        for c in idx_copies:
            c.wait()

        gathers = [None] * n_chunks
        writes = [None] * n_chunks
        for j in range(n_chunks + 1):
            if j < n_chunks:
                buf = j % n_buf
                if j >= n_buf:
                    writes[j - n_buf].wait()
                gathers[j] = pltpu.async_copy(
                    table_hbm.at[idx_v.at[j]], rows_v.at[buf], gsem.at[buf]
                )
            if j >= 1:
                jj = j - 1
                buf = jj % n_buf
                gathers[jj].wait()
                writes[jj] = pltpu.async_copy(
                    rows_v.at[buf],
                    out_hbm.at[pl.ds(base + jj * chunk, chunk)],
                    wsem.at[buf],
                )
        for jj in range(n_chunks - n_buf, n_chunks):
            writes[jj].wait()

    return gather_kernel


_gather = _make_gather()


@jax.jit
def kernel(inp_ids, table):
    flat = inp_ids.reshape(B)
    out = _gather(flat, table)
    return out.reshape(BATCH, SEQ, EMBED_DIM)
